# baseline (speedup 1.0000x reference)
import jax
import jax.numpy as jnp
from jax import lax
from jax.experimental import pallas as pl
from jax.experimental.pallas import tpu as pltpu
from jax.experimental.pallas import tpu_sc as plsc

_NUM_CORES = 2
_NUM_SUBCORES = 16
_NUM_WORKERS = _NUM_CORES * _NUM_SUBCORES

_GROUP = 128


def _gather_call(n_idx, dim, idx_dtype):
    assert n_idx % (_GROUP * _NUM_WORKERS) == 0
    groups_per_worker = n_idx // (_GROUP * _NUM_WORKERS)

    mesh = plsc.VectorSubcoreMesh(core_axis_name="c", subcore_axis_name="s")

    def body(idx_hbm, table_hbm, out_hbm, idx_v, buf, gsem):
        wid = lax.axis_index("s") * _NUM_CORES + lax.axis_index("c")
        gbase = wid * groups_per_worker
        pltpu.sync_copy(idx_hbm.at[pl.ds(gbase, groups_per_worker)], idx_v)

        def step(g, carry):
            pltpu.async_copy(table_hbm.at[idx_v.at[g]], buf, gsem).wait()
            pltpu.sync_copy(
                buf, out_hbm.at[pl.ds((gbase + g) * _GROUP, _GROUP)]
            )
            return carry

        lax.fori_loop(0, groups_per_worker, step, 0)

    return pl.kernel(
        body,
        out_type=jax.ShapeDtypeStruct((n_idx, dim), jnp.float32),
        mesh=mesh,
        scratch_types=[
            pltpu.VMEM((groups_per_worker, _GROUP), idx_dtype),
            pltpu.VMEM((_GROUP, dim), jnp.float32),
            pltpu.SemaphoreType.DMA,
        ],
        compiler_params=pltpu.CompilerParams(use_tc_tiling_on_sc=False),
    )


def kernel(inputs, matrix):
    batch, hist = inputs.shape
    _, dim = matrix.shape
    n_idx = batch * hist
    idx2d = inputs.reshape(n_idx // _GROUP, _GROUP)
    f = _gather_call(n_idx, dim, idx2d.dtype)
    out = f(idx2d, matrix)
    return out.reshape(batch, hist, dim)


# speedup vs baseline: 1.1110x; 1.1110x over previous
import jax
import jax.numpy as jnp
from jax import lax
from jax.experimental import pallas as pl
from jax.experimental.pallas import tpu as pltpu
from jax.experimental.pallas import tpu_sc as plsc

_NUM_CORES = 2
_NUM_SUBCORES = 16
_NUM_WORKERS = _NUM_CORES * _NUM_SUBCORES

_GROUP = 128


def _gather_call(n_idx, dim, idx_dtype):
    assert n_idx % (_GROUP * _NUM_WORKERS) == 0
    groups_per_worker = n_idx // (_GROUP * _NUM_WORKERS)

    mesh = plsc.VectorSubcoreMesh(core_axis_name="c", subcore_axis_name="s")

    half = 4
    n_pairs = groups_per_worker // (2 * half)
    assert groups_per_worker % (2 * half) == 0

    def body(idx_hbm, table_hbm, out_hbm, idx_v, bufs, gsem, wsem):
        wid = lax.axis_index("s") * _NUM_CORES + lax.axis_index("c")
        gbase = wid * groups_per_worker
        pltpu.sync_copy(idx_hbm.at[pl.ds(gbase, groups_per_worker)], idx_v)

        def start_gathers(h, g0):
            for b in range(half):
                pltpu.async_copy(
                    table_hbm.at[idx_v.at[g0 + b]],
                    bufs.at[h * half + b],
                    gsem,
                )

        def drain_gathers():
            for b in range(half):
                pltpu.make_async_copy(
                    table_hbm.at[pl.ds(0, _GROUP)], bufs.at[b], gsem
                ).wait()

        def start_writes(h, g0):
            for b in range(half):
                pltpu.async_copy(
                    bufs.at[h * half + b],
                    out_hbm.at[pl.ds((gbase + g0 + b) * _GROUP, _GROUP)],
                    wsem,
                )

        def drain_writes():
            for b in range(half):
                pltpu.make_async_copy(
                    bufs.at[b], out_hbm.at[pl.ds(0, _GROUP)], wsem
                ).wait()

        start_gathers(0, 0)

        def pair(u, carry):
            g0a = u * 2 * half
            g0b = g0a + half
            drain_gathers()
            start_gathers(1, g0b)
            start_writes(0, g0a)
            drain_writes()
            drain_gathers()

            @pl.when(u < n_pairs - 1)
            def _():
                start_gathers(0, g0a + 2 * half)

            start_writes(1, g0b)
            drain_writes()
            return carry

        lax.fori_loop(0, n_pairs, pair, 0)

    return pl.kernel(
        body,
        out_type=jax.ShapeDtypeStruct((n_idx, dim), jnp.float32),
        mesh=mesh,
        scratch_types=[
            pltpu.VMEM((groups_per_worker, _GROUP), idx_dtype),
            pltpu.VMEM((2 * half, _GROUP, dim), jnp.float32),
            pltpu.SemaphoreType.DMA,
            pltpu.SemaphoreType.DMA,
        ],
        compiler_params=pltpu.CompilerParams(use_tc_tiling_on_sc=False),
    )


def kernel(inputs, matrix):
    batch, hist = inputs.shape
    _, dim = matrix.shape
    n_idx = batch * hist
    idx2d = inputs.reshape(n_idx // _GROUP, _GROUP)
    f = _gather_call(n_idx, dim, idx2d.dtype)
    out = f(idx2d, matrix)
    return out.reshape(batch, hist, dim)
